# Initial kernel scaffold; baseline (speedup 1.0000x reference)
import functools

import jax
import jax.numpy as jnp
from jax import lax
from jax.experimental import pallas as pl
from jax.experimental.pallas import tpu as pltpu
from jax.experimental.pallas import tpu_sc as plsc

NC = 2
NS = 16
L = 16
NW = NC * NS


def _mm_body(x_ref, w_ref, o_ref):
    o_ref[...] = jnp.dot(x_ref[...], w_ref[...],
                         preferred_element_type=jnp.float32)


def _matmul(x, w):
    n, d = x.shape
    bn = 2000
    return pl.pallas_call(
        _mm_body,
        grid=(n // bn,),
        in_specs=[
            pl.BlockSpec((bn, d), lambda i: (i, 0)),
            pl.BlockSpec((d, d), lambda i: (0, 0)),
        ],
        out_specs=pl.BlockSpec((bn, d), lambda i: (i, 0)),
        out_shape=jax.ShapeDtypeStruct((n, d), jnp.float32),
    )(x, w)


def _comb_body(p0_ref, p1_ref, b_ref, o_ref):
    o_ref[...] = p0_ref[...] + p1_ref[...] + b_ref[...]


def _combine(p0, p1, bias2d):
    n, d = p0.shape
    bn = 2000
    return pl.pallas_call(
        _comb_body,
        grid=(n // bn,),
        in_specs=[
            pl.BlockSpec((bn, d), lambda i: (i, 0)),
            pl.BlockSpec((bn, d), lambda i: (i, 0)),
            pl.BlockSpec((1, d), lambda i: (0, 0)),
        ],
        out_specs=pl.BlockSpec((bn, d), lambda i: (i, 0)),
        out_shape=jax.ShapeDtypeStruct((n, d), jnp.float32),
    )(p0, p1, bias2d)


def _sc_scatter(support, src, dst, rel, alpha_flat):
    n, d = support.shape
    e = src.shape[0]
    apad = alpha_flat.shape[0]
    assert e % NW == 0
    epw = e // NW
    c = 80
    assert epw % c == 0
    chunks = epw // c
    assert n % NS == 0
    rpt = n // NS
    zrows = 125
    assert rpt % zrows == 0
    nvec = d // L

    mesh = plsc.VectorSubcoreMesh(core_axis_name="c", subcore_axis_name="s")

    @functools.partial(
        pl.kernel,
        out_type=jax.ShapeDtypeStruct((NC * n, d), jnp.float32),
        mesh=mesh,
        scratch_types=[
            pltpu.VMEM((apad,), jnp.float32),
            pltpu.VMEM((c,), jnp.int32),
            pltpu.VMEM((c,), jnp.int32),
            pltpu.VMEM((c,), jnp.int32),
            pltpu.VMEM((c, d), jnp.float32),
            pltpu.VMEM((c, d), jnp.float32),
            pltpu.VMEM((125, d), jnp.float32),
            pltpu.VMEM_SHARED((n, d), jnp.float32),
            pltpu.SemaphoreType.DMA,
            pltpu.SemaphoreType.DMA,
        ],
    )
    def k(support_hbm, src_hbm, dst_hbm, rel_hbm, alpha_hbm, out_hbm,
          alpha_v, src_v, dst_v, rel_v, rows_d, rows_s, zbuf, acc,
          sem_d, sem_s):
        cid = lax.axis_index("c")
        sid = lax.axis_index("s")
        wid = cid * NS + sid

        pltpu.sync_copy(alpha_hbm, alpha_v)

        zero16 = jnp.zeros((L,), jnp.float32)

        def zrow(i, carry):
            for v in range(nvec):
                zbuf[i, pl.ds(v * L, L)] = zero16
            return carry

        lax.fori_loop(0, zrows, zrow, 0)

        def zcp(g, carry):
            pltpu.sync_copy(
                zbuf, acc.at[pl.ds(sid * rpt + g * zrows, zrows)])
            return carry

        lax.fori_loop(0, rpt // zrows, zcp, 0)
        plsc.subcore_barrier()

        base0 = wid * epw

        def chunk(g, carry):
            base = base0 + g * c
            pltpu.sync_copy(src_hbm.at[pl.ds(base, c)], src_v)
            pltpu.sync_copy(dst_hbm.at[pl.ds(base, c)], dst_v)
            pltpu.sync_copy(rel_hbm.at[pl.ds(base, c)], rel_v)
            cp_d = pltpu.async_copy(support_hbm.at[dst_v], rows_d, sem_d)
            cp_s = pltpu.async_copy(support_hbm.at[src_v], rows_s, sem_s)
            cp_d.wait()
            cp_s.wait()

            def edge(ei, icarry):
                r = rel_v[ei]
                a = alpha_v[r]
                av = jnp.full((L,), a, jnp.float32)
                for v in range(nvec):
                    sl = pl.ds(v * L, L)
                    rows_d[ei, sl] = rows_d[ei, sl] * av
                    rows_s[ei, sl] = rows_s[ei, sl] * av
                return icarry

            lax.fori_loop(0, c, edge, 0)
            pltpu.sync_copy(rows_d, acc.at[src_v], add=True)
            pltpu.sync_copy(rows_s, acc.at[dst_v], add=True)
            return carry

        lax.fori_loop(0, chunks, chunk, 0)
        plsc.subcore_barrier()

        def ocp(g, carry):
            off = sid * rpt + g * zrows
            pltpu.sync_copy(acc.at[pl.ds(off, zrows)],
                            out_hbm.at[pl.ds(cid * n + off, zrows)])
            return carry

        lax.fori_loop(0, rpt // zrows, ocp, 0)

    return k(support, src, dst, rel, alpha_flat)


def kernel(input, edge_index, rel_type, n_nodes, weight, bias, alpha):
    n, d = input.shape
    support = _matmul(input, weight)
    src = edge_index[0].astype(jnp.int32)
    dst = edge_index[1].astype(jnp.int32)
    rel = rel_type.astype(jnp.int32)
    apad = 512
    alpha_flat = jnp.zeros((apad,), jnp.float32).at[: alpha.shape[0]].set(
        alpha[:, 0])
    partials = _sc_scatter(support, src, dst, rel, alpha_flat)
    return _combine(partials[:n], partials[n:], bias[None, :])


# speedup vs baseline: 9.7356x; 9.7356x over previous
import functools

import jax
import jax.numpy as jnp
from jax import lax
from jax.experimental import pallas as pl
from jax.experimental.pallas import tpu as pltpu
from jax.experimental.pallas import tpu_sc as plsc

NC = 2
NS = 16
L = 16
NW = NC * NS


def _mm_body(x_ref, w_ref, o_ref):
    o_ref[...] = jnp.dot(x_ref[...], w_ref[...],
                         preferred_element_type=jnp.float32)


def _matmul(x, w):
    n, d = x.shape
    bn = 2000
    return pl.pallas_call(
        _mm_body,
        grid=(n // bn,),
        in_specs=[
            pl.BlockSpec((bn, d), lambda i: (i, 0)),
            pl.BlockSpec((d, d), lambda i: (0, 0)),
        ],
        out_specs=pl.BlockSpec((bn, d), lambda i: (i, 0)),
        out_shape=jax.ShapeDtypeStruct((n, d), jnp.float32),
    )(x, w)


def _comb_body(p0_ref, p1_ref, b_ref, o_ref):
    o_ref[...] = p0_ref[...] + p1_ref[...] + b_ref[...]


def _combine(p0, p1, bias2d):
    n, d = p0.shape
    bn = 2000
    return pl.pallas_call(
        _comb_body,
        grid=(n // bn,),
        in_specs=[
            pl.BlockSpec((bn, d), lambda i: (i, 0)),
            pl.BlockSpec((bn, d), lambda i: (i, 0)),
            pl.BlockSpec((1, d), lambda i: (0, 0)),
        ],
        out_specs=pl.BlockSpec((bn, d), lambda i: (i, 0)),
        out_shape=jax.ShapeDtypeStruct((n, d), jnp.float32),
    )(p0, p1, bias2d)


def _sc_scatter(support, src, dst, rel, alpha_flat):
    n, d = support.shape
    e = src.shape[0]
    apad = alpha_flat.shape[0]
    assert e % NW == 0
    epw = e // NW
    c = 80
    assert epw % c == 0
    chunks = epw // c
    zrows = 128
    npad = ((n + NS * zrows - 1) // (NS * zrows)) * (NS * zrows)
    rpt = npad // NS
    assert rpt % zrows == 0
    nvec = d // L

    mesh = plsc.VectorSubcoreMesh(core_axis_name="c", subcore_axis_name="s")

    @functools.partial(
        pl.kernel,
        out_type=jax.ShapeDtypeStruct((NC * npad, d), jnp.float32),
        mesh=mesh,
        scratch_types=[
            pltpu.VMEM((c,), jnp.float32),
            pltpu.VMEM((c,), jnp.int32),
            pltpu.VMEM((c,), jnp.int32),
            pltpu.VMEM((c,), jnp.int32),
            pltpu.VMEM((c, d), jnp.float32),
            pltpu.VMEM((c, d), jnp.float32),
            pltpu.VMEM((zrows, d), jnp.float32),
            pltpu.VMEM_SHARED((npad, d), jnp.float32),
            pltpu.SemaphoreType.DMA,
            pltpu.SemaphoreType.DMA,
            pltpu.SemaphoreType.DMA,
        ],
    )
    def k(support_hbm, src_hbm, dst_hbm, rel_hbm, alpha_hbm, out_hbm,
          alp_v, src_v, dst_v, rel_v, rows_d, rows_s, zbuf, acc,
          sem_d, sem_s, sem_a):
        cid = lax.axis_index("c")
        sid = lax.axis_index("s")
        wid = cid * NS + sid

        zero16 = jnp.zeros((L,), jnp.float32)

        def zrow(i, carry):
            for v in range(nvec):
                zbuf[i, pl.ds(v * L, L)] = zero16
            return carry

        lax.fori_loop(0, zrows, zrow, 0)

        def zcp(g, carry):
            pltpu.sync_copy(
                zbuf, acc.at[pl.ds(sid * rpt + g * zrows, zrows)])
            return carry

        lax.fori_loop(0, rpt // zrows, zcp, 0)
        plsc.subcore_barrier()

        base0 = wid * epw

        def chunk(g, carry):
            base = base0 + g * c
            pltpu.sync_copy(src_hbm.at[pl.ds(base, c)], src_v)
            pltpu.sync_copy(dst_hbm.at[pl.ds(base, c)], dst_v)
            pltpu.sync_copy(rel_hbm.at[pl.ds(base, c)], rel_v)
            cp_d = pltpu.async_copy(support_hbm.at[dst_v], rows_d, sem_d)
            cp_s = pltpu.async_copy(support_hbm.at[src_v], rows_s, sem_s)
            cp_a = pltpu.async_copy(alpha_hbm.at[rel_v], alp_v, sem_a)
            cp_d.wait()
            cp_s.wait()
            cp_a.wait()

            def group(j, icarry):
                e0 = j * L
                a16 = alp_v[pl.ds(e0, L)]
                for kk in range(L):
                    av = jnp.full((L,), a16[kk], jnp.float32)
                    ei = e0 + kk
                    for v in range(nvec):
                        sl = pl.ds(v * L, L)
                        rows_d[ei, sl] = rows_d[ei, sl] * av
                        rows_s[ei, sl] = rows_s[ei, sl] * av
                return icarry

            lax.fori_loop(0, c // L, group, 0)
            pltpu.sync_copy(rows_d, acc.at[src_v], add=True)
            pltpu.sync_copy(rows_s, acc.at[dst_v], add=True)
            return carry

        lax.fori_loop(0, chunks, chunk, 0)
        plsc.subcore_barrier()

        def ocp(g, carry):
            off = sid * rpt + g * zrows
            pltpu.sync_copy(acc.at[pl.ds(off, zrows)],
                            out_hbm.at[pl.ds(cid * npad + off, zrows)])
            return carry

        lax.fori_loop(0, rpt // zrows, ocp, 0)

    return k(support, src, dst, rel, alpha_flat)


def partials_rows(n):
    return ((n + NS * 128 - 1) // (NS * 128)) * (NS * 128)


def kernel(input, edge_index, rel_type, n_nodes, weight, bias, alpha):
    n, d = input.shape
    support = _matmul(input, weight)
    src = edge_index[0].astype(jnp.int32)
    dst = edge_index[1].astype(jnp.int32)
    rel = rel_type.astype(jnp.int32)
    apad = 512
    alpha_flat = jnp.zeros((apad,), jnp.float32).at[: alpha.shape[0]].set(
        alpha[:, 0])
    npad = partials_rows(n)
    partials = _sc_scatter(support, src, dst, rel, alpha_flat)
    return _combine(partials[:n], partials[npad:npad + n], bias[None, :])


# speedup vs baseline: 10.3283x; 1.0609x over previous
import functools

import jax
import jax.numpy as jnp
from jax import lax
from jax.experimental import pallas as pl
from jax.experimental.pallas import tpu as pltpu
from jax.experimental.pallas import tpu_sc as plsc

NC = 2
NS = 16
L = 16
NW = NC * NS


def _mm_body(x_ref, w_ref, o_ref):
    o_ref[...] = jnp.dot(x_ref[...], w_ref[...],
                         preferred_element_type=jnp.float32)


def _matmul(x, w):
    n, d = x.shape
    bn = 2000
    return pl.pallas_call(
        _mm_body,
        grid=(n // bn,),
        in_specs=[
            pl.BlockSpec((bn, d), lambda i: (i, 0)),
            pl.BlockSpec((d, d), lambda i: (0, 0)),
        ],
        out_specs=pl.BlockSpec((bn, d), lambda i: (i, 0)),
        out_shape=jax.ShapeDtypeStruct((n, d), jnp.float32),
    )(x, w)


def _comb_body(p0_ref, p1_ref, b_ref, o_ref):
    o_ref[...] = p0_ref[...] + p1_ref[...] + b_ref[...]


def _combine(p0, p1, bias2d):
    n, d = p0.shape
    bn = 2000
    return pl.pallas_call(
        _comb_body,
        grid=(n // bn,),
        in_specs=[
            pl.BlockSpec((bn, d), lambda i: (i, 0)),
            pl.BlockSpec((bn, d), lambda i: (i, 0)),
            pl.BlockSpec((1, d), lambda i: (0, 0)),
        ],
        out_specs=pl.BlockSpec((bn, d), lambda i: (i, 0)),
        out_shape=jax.ShapeDtypeStruct((n, d), jnp.float32),
    )(p0, p1, bias2d)


def _sc_scatter(support, src, dst, rel, alpha_flat, zeros_rows):
    n, d = support.shape
    e = src.shape[0]
    assert e % NW == 0
    epw = e // NW
    c = 80
    nbuf = 2
    chunks = epw // c
    assert epw % c == 0
    outer = chunks // nbuf
    peeled = chunks - outer * nbuf
    npad = ((n + NS * 128 - 1) // (NS * 128)) * (NS * 128)
    rpt = npad // NS
    assert zeros_rows.shape == (rpt, d)
    nvec = d // L

    mesh = plsc.VectorSubcoreMesh(core_axis_name="c", subcore_axis_name="s")

    buf_types = []
    for _ in range(nbuf):
        buf_types += [
            pltpu.VMEM((c,), jnp.float32),
            pltpu.VMEM((c,), jnp.int32),
            pltpu.VMEM((c,), jnp.int32),
            pltpu.VMEM((c,), jnp.int32),
            pltpu.VMEM((c, d), jnp.float32),
            pltpu.VMEM((c, d), jnp.float32),
            pltpu.SemaphoreType.DMA,
            pltpu.SemaphoreType.DMA,
            pltpu.SemaphoreType.DMA,
        ]

    @functools.partial(
        pl.kernel,
        out_type=jax.ShapeDtypeStruct((NC * npad, d), jnp.float32),
        mesh=mesh,
        scratch_types=[
            pltpu.VMEM_SHARED((npad, d), jnp.float32),
        ] + buf_types,
    )
    def k(support_hbm, src_hbm, dst_hbm, rel_hbm, alpha_hbm, zeros_hbm,
          out_hbm, acc, *bufs):
        alp_v = [bufs[9 * b + 0] for b in range(nbuf)]
        src_v = [bufs[9 * b + 1] for b in range(nbuf)]
        dst_v = [bufs[9 * b + 2] for b in range(nbuf)]
        rel_v = [bufs[9 * b + 3] for b in range(nbuf)]
        rows_d = [bufs[9 * b + 4] for b in range(nbuf)]
        rows_s = [bufs[9 * b + 5] for b in range(nbuf)]
        sem_i = [bufs[9 * b + 6] for b in range(nbuf)]
        sem_g = [bufs[9 * b + 7] for b in range(nbuf)]
        sem_sc = [bufs[9 * b + 8] for b in range(nbuf)]

        cid = lax.axis_index("c")
        sid = lax.axis_index("s")
        wid = cid * NS + sid
        base0 = wid * epw

        def issue_idx(chunk_i, b):
            base = base0 + chunk_i * c
            pltpu.async_copy(src_hbm.at[pl.ds(base, c)], src_v[b], sem_i[b])
            pltpu.async_copy(dst_hbm.at[pl.ds(base, c)], dst_v[b], sem_i[b])
            pltpu.async_copy(rel_hbm.at[pl.ds(base, c)], rel_v[b], sem_i[b])

        def wait_idx(b):
            for ref in (src_v[b], dst_v[b], rel_v[b]):
                pltpu.make_async_copy(
                    src_hbm.at[pl.ds(0, c)], ref, sem_i[b]).wait()

        def issue_gather(b):
            return (
                pltpu.async_copy(
                    support_hbm.at[dst_v[b]], rows_d[b], sem_g[b]),
                pltpu.async_copy(
                    support_hbm.at[src_v[b]], rows_s[b], sem_g[b]),
                pltpu.async_copy(
                    alpha_hbm.at[rel_v[b]], alp_v[b], sem_g[b]),
            )

        def issue_scatter(b):
            return (
                pltpu.async_copy(
                    rows_d[b], acc.at[src_v[b]], sem_sc[b], add=True),
                pltpu.async_copy(
                    rows_s[b], acc.at[dst_v[b]], sem_sc[b], add=True),
            )

        def compute(b):
            def group(j, icarry):
                e0 = j * L
                a16 = alp_v[b][pl.ds(e0, L)]
                for kk in range(L):
                    av = jnp.full((L,), a16[kk], jnp.float32)
                    ei = e0 + kk
                    for v in range(nvec):
                        sl = pl.ds(v * L, L)
                        rows_d[b][ei, sl] = rows_d[b][ei, sl] * av
                        rows_s[b][ei, sl] = rows_s[b][ei, sl] * av
                return icarry

            lax.fori_loop(0, c // L, group, 0)

        for b in range(nbuf):
            issue_idx(b, b)
        pltpu.sync_copy(zeros_hbm, acc.at[pl.ds(sid * rpt, rpt)])
        plsc.subcore_barrier()

        def outer_body(o, carry):
            gcps = []
            for b in range(nbuf):
                wait_idx(b)
                gcps.append(issue_gather(b))
            scps = []
            for b in range(nbuf):
                for g in gcps[b]:
                    g.wait()
                compute(b)
                scps.append(issue_scatter(b))
            for b in range(nbuf):
                for s in scps[b]:
                    s.wait()

                @pl.when(o + 1 < outer)
                def _():
                    issue_idx((o + 1) * nbuf + b, b)

            return carry

        lax.fori_loop(0, outer, outer_body, 0)

        for p in range(peeled):
            issue_idx(outer * nbuf + p, 0)
            wait_idx(0)
            for g in issue_gather(0):
                g.wait()
            compute(0)
            for s in issue_scatter(0):
                s.wait()

        plsc.subcore_barrier()

        pltpu.sync_copy(acc.at[pl.ds(sid * rpt, rpt)],
                        out_hbm.at[pl.ds(cid * npad + sid * rpt, rpt)])

    return k(support, src, dst, rel, alpha_flat, zeros_rows)


def partials_rows(n):
    return ((n + NS * 128 - 1) // (NS * 128)) * (NS * 128)


def kernel(input, edge_index, rel_type, n_nodes, weight, bias, alpha):
    n, d = input.shape
    support = _matmul(input, weight)
    src = edge_index[0].astype(jnp.int32)
    dst = edge_index[1].astype(jnp.int32)
    rel = rel_type.astype(jnp.int32)
    apad = 512
    alpha_flat = jnp.zeros((apad,), jnp.float32).at[: alpha.shape[0]].set(
        alpha[:, 0])
    npad = partials_rows(n)
    zeros_rows = jnp.zeros((npad // NS, d), jnp.float32)
    partials = _sc_scatter(support, src, dst, rel, alpha_flat, zeros_rows)
    return _combine(partials[:n], partials[npad:npad + n], bias[None, :])
